# Initial kernel scaffold; baseline (speedup 1.0000x reference)
import jax
import jax.numpy as jnp
from jax.experimental import pallas as pl


def kernel(feat, W1, Wf, b1, W2, b2, W3, b3):
    raise NotImplementedError("write your pallas kernel here")


# speedup vs baseline: 1.0555x; 1.0555x over previous
import functools

import jax
import jax.numpy as jnp
import numpy as np
from jax import lax
from jax.experimental import pallas as pl
from jax.experimental.pallas import tpu as pltpu
from jax.experimental.pallas import tpu_sc as plsc

BZ = 8
HID = 256
K = 8192
RES_LAST = 513
KEY_MAX = 0x3F000000
NC, NS, L = 2, 16, 16

LEVELS = [(65, 128), (129, 256), (257, 384), (513, 640)]


def _up_mats(rp, wpp, r, wp):
    m = np.zeros((r, rp), np.float32)
    i = np.arange(rp)
    m[2 * i, i] = 1.0
    i = np.arange(rp - 1)
    m[2 * i + 1, i] = 0.5
    m[2 * i + 1, i + 1] = 0.5
    mt = np.zeros((wpp, wp), np.float32)
    mt[:rp, :r] = m.T
    return jnp.asarray(m), jnp.asarray(mt)


def _mk_up_kernel(rp, wpp, r, wp):
    rq = -(-r // 4)

    def body(prev_ref, m_ref, mt_ref, occ_ref, keys_ref, stats_ref):
        prev = prev_ref[0]
        tmp = jnp.dot(m_ref[:, :], prev, preferred_element_type=jnp.float32)
        out = jnp.dot(tmp, mt_ref[:, :], preferred_element_type=jnp.float32)
        occ_ref[0] = out
        a = jnp.abs(out - 0.5)
        key = KEY_MAX - lax.bitcast_convert_type(a, jnp.int32)
        col = lax.broadcasted_iota(jnp.int32, (r, wp), 1)
        key = jnp.where(col < r, key, -1)
        keys_ref[0] = key

        def bs(_, lohi):
            lo, hi = lohi
            mid = lo + (hi - lo + 1) // 2
            cnt = jnp.sum((key >= mid).astype(jnp.int32))
            ge = cnt >= K
            return (jnp.where(ge, mid, lo), jnp.where(ge, hi, mid - 1))

        lo, _ = lax.fori_loop(
            0, 31, bs, (jnp.int32(0), jnp.int32(KEY_MAX + 1)))
        t = lo
        mgt = (key > t).astype(jnp.int32)
        meq = (key == t).astype(jnp.int32)
        cg = [jnp.sum(mgt[q * rq:min((q + 1) * rq, r)]) for q in range(4)]
        ce = [jnp.sum(meq[q * rq:min((q + 1) * rq, r)]) for q in range(4)]
        c_gt = cg[0] + cg[1] + cg[2] + cg[3]
        m_take = K - c_gt
        vals = [t, m_take] + cg + ce + [jnp.int32(0)] * 6
        ii = lax.broadcasted_iota(jnp.int32, (1, 16), 1)
        acc = jnp.zeros((1, 16), jnp.int32)
        for e, v in enumerate(vals):
            acc = jnp.where(ii == e, v, acc)
        stats_ref[0] = acc

    def call(occ_prev, m, mt):
        return pl.pallas_call(
            body,
            grid=(BZ,),
            in_specs=[
                pl.BlockSpec((1, rp, wpp), lambda b: (b, 0, 0)),
                pl.BlockSpec((r, rp), lambda b: (0, 0)),
                pl.BlockSpec((wpp, wp), lambda b: (0, 0)),
            ],
            out_specs=[
                pl.BlockSpec((1, r, wp), lambda b: (b, 0, 0)),
                pl.BlockSpec((1, r, wp), lambda b: (b, 0, 0)),
                pl.BlockSpec((1, 1, 16), lambda b: (b, 0, 0)),
            ],
            out_shape=[
                jax.ShapeDtypeStruct((BZ, r, wp), jnp.float32),
                jax.ShapeDtypeStruct((BZ, r, wp), jnp.int32),
                jax.ShapeDtypeStruct((BZ, 1, 16), jnp.int32),
            ],
        )(occ_prev, m, mt)

    return call


def _mk_mlp_kernel(r, stride):
    half_step = float(np.float32(np.float32(1.0 / RES_LAST) / 2))

    def body(idx_ref, feat_ref, w1_ref, wf_ref, b1_ref, w2_ref, b2_ref,
             w3_ref, b3_ref, out_ref):
        idx = idx_ref[0]
        px = (idx % r).astype(jnp.float32) * stride
        py = (idx // r).astype(jnp.float32) * stride
        cx = (px / 513.0 + half_step) * 2.0 - 1.0
        cy = (py / 513.0 + half_step) * 2.0 - 1.0
        fw = jnp.dot(feat_ref[0], wf_ref[:, :],
                     preferred_element_type=jnp.float32) + b1_ref[:, :]
        h1 = jnp.maximum(cx * w1_ref[0:1, :] + cy * w1_ref[1:2, :] + fw, 0.0)
        h2 = jnp.maximum(
            jnp.dot(h1, w2_ref[:, :], preferred_element_type=jnp.float32)
            + b2_ref[:, :], 0.0)
        z = jnp.sum(h2 * w3_ref[:, :], axis=1, keepdims=True) + b3_ref[:, :]
        out_ref[0] = jax.nn.sigmoid(z)

    TK = 2048

    def call(idx3, feat, w1, wf, b1r, w2, b2r, w3r, b3r):
        return pl.pallas_call(
            body,
            grid=(BZ, K // TK),
            in_specs=[
                pl.BlockSpec((1, TK, 1), lambda b, t: (b, t, 0)),
                pl.BlockSpec((1, 1, HID), lambda b, t: (b, 0, 0)),
                pl.BlockSpec((2, HID), lambda b, t: (0, 0)),
                pl.BlockSpec((HID, HID), lambda b, t: (0, 0)),
                pl.BlockSpec((1, HID), lambda b, t: (0, 0)),
                pl.BlockSpec((HID, HID), lambda b, t: (0, 0)),
                pl.BlockSpec((1, HID), lambda b, t: (0, 0)),
                pl.BlockSpec((1, HID), lambda b, t: (0, 0)),
                pl.BlockSpec((1, 1), lambda b, t: (0, 0)),
            ],
            out_specs=pl.BlockSpec((1, TK, 1), lambda b, t: (b, t, 0)),
            out_shape=jax.ShapeDtypeStruct((BZ, K, 1), jnp.float32),
        )(idx3, feat.reshape(BZ, 1, HID), w1, wf, b1r, w2, b2r, w3r, b3r)

    return call


def _mk_compact_kernel(r, wp):
    rq = -(-r // 4)
    tot = r * wp
    nsteps = wp // L
    mesh = plsc.VectorSubcoreMesh(core_axis_name="c", subcore_axis_name="s")

    def body(keys_hbm, stats_hbm, out_hbm, kbuf, gtbuf, eqbuf, statv,
             offs_ref, sem):
        cid = lax.axis_index("c")
        sid = lax.axis_index("s")
        wid = cid * NS + sid
        b = lax.div(wid, 4)
        q = lax.rem(wid, 4)
        lane = lax.iota(jnp.int32, 16)

        pltpu.sync_copy(stats_hbm.at[b], statv)
        sv = statv[...]

        def ssum(msk):
            return jnp.sum(jnp.where(msk, sv, jnp.zeros_like(sv)))

        t = ssum(lane == 0)
        m_take = ssum(lane == 1)
        cgt_q = ssum(lane == 2 + q)
        gt_before = ssum((lane >= 2) & (lane < 2 + q))
        c_gt_tot = ssum((lane >= 2) & (lane < 6))
        ceq_q = ssum(lane == 6 + q)
        eq_before = ssum((lane >= 6) & (lane < 6 + q))
        eq_take = jnp.clip(m_take - eq_before, 0, ceq_q)

        tv = jnp.full((16,), t, jnp.int32)
        capv = jnp.full((16,), eq_take, jnp.int32)
        r0 = q * rq
        r1 = jnp.minimum(r0 + rq, r)
        base = b * tot

        def row(rr, carry):
            gt_off, eq_off = carry
            pltpu.sync_copy(keys_hbm.at[pl.ds(base + rr * wp, wp)], kbuf)
            gbase = jnp.full((16,), rr * r, jnp.int32) + lane
            for j in range(nsteps):
                kv = kbuf[pl.ds(j * L, L)]
                gvec = gbase + (j * L)
                mgt = kv > tv
                pgt = plsc.cumsum(mgt.astype(jnp.int32))
                plsc.store_scatter(gtbuf, [gt_off + pgt - 1], gvec, mask=mgt)
                gt_off = gt_off + plsc.all_reduce_population_count(mgt)
                meq = kv == tv
                peq = plsc.cumsum(meq.astype(jnp.int32))
                pose = eq_off + peq - 1
                plsc.store_scatter(eqbuf, [pose], gvec,
                                   mask=meq & (pose < capv))
                eq_off = eq_off + plsc.all_reduce_population_count(meq)
            return gt_off, eq_off

        z16 = jnp.zeros((16,), jnp.int32)
        lax.fori_loop(r0, r1, row, (z16, z16))

        def write_list(buf, cnt, gstart):
            nch = lax.div(cnt + 127, 128)

            def chunk(ci, _):
                for jj in range(8):
                    pos = jnp.full((16,), gstart + ci * 128 + jj * L,
                                   jnp.int32) + lane
                    pos = jnp.where(pos < gstart + cnt, pos, BZ * K + lane)
                    offs_ref[0, pl.ds(jj * L, L)] = pos
                cp = pltpu.async_copy(buf.at[pl.ds(ci * 128, 128)],
                                      out_hbm.at[offs_ref.at[0]], sem)
                cp.wait()
                return 0

            lax.fori_loop(0, nch, chunk, 0)

        write_list(gtbuf, cgt_q, b * K + gt_before)
        write_list(eqbuf, eq_take,
                   b * K + c_gt_tot + jnp.minimum(eq_before, m_take))

    def call(keys_flat, stats2):
        return pl.kernel(
            body,
            out_type=jax.ShapeDtypeStruct((BZ * K + 128,), jnp.int32),
            mesh=mesh,
            scratch_types=[
                pltpu.VMEM((wp,), jnp.int32),
                pltpu.VMEM((K + 16,), jnp.int32),
                pltpu.VMEM((K + 16,), jnp.int32),
                pltpu.VMEM((16,), jnp.int32),
                pltpu.VMEM((1, 128), jnp.int32),
                pltpu.SemaphoreType.DMA,
            ],
            compiler_params=pltpu.CompilerParams(needs_layout_passes=False),
        )(keys_flat, stats2)

    return call


def _mk_scatter_kernel(r, wp, has_invalid):
    tot = r * wp
    share = tot // 4
    ch2 = min(8192, share)
    n_full, rem = divmod(share, ch2)
    seg = K // 4

    def body(occ_in, idx_hbm, vals_hbm, occ_out, cbuf, ibuf, vbuf, offs_ref,
             sem):
        cid = lax.axis_index("c")
        sid = lax.axis_index("s")
        my_off = cid * (4 * tot) + sid * share
        for i in range(n_full):
            pltpu.sync_copy(occ_in.at[pl.ds(my_off + i * ch2, ch2)], cbuf)
            pltpu.sync_copy(cbuf, occ_out.at[pl.ds(my_off + i * ch2, ch2)])
        if rem:
            o2 = my_off + n_full * ch2
            pltpu.sync_copy(occ_in.at[pl.ds(o2, rem)], cbuf.at[pl.ds(0, rem)])
            pltpu.sync_copy(cbuf.at[pl.ds(0, rem)], occ_out.at[pl.ds(o2, rem)])
        plsc.subcore_barrier()

        b = cid * 4 + lax.div(sid, 4)
        qq = lax.rem(sid, 4)
        soff = b * K + qq * seg
        pltpu.sync_copy(idx_hbm.at[pl.ds(soff, seg)], ibuf)
        pltpu.sync_copy(vals_hbm.at[pl.ds(soff, seg)], vbuf)
        lane = lax.iota(jnp.int32, 16)
        rsplat = jnp.full((16,), r, jnp.int32)

        def chunk(ci, _):
            for jj in range(8):
                g = ibuf[pl.ds(ci * 128 + jj * L, L)]
                y = lax.div(g, rsplat)
                o = g + y * (wp - r) + b * tot
                if has_invalid:
                    o = jnp.where(g < r * r, o, b * tot + wp - 1)
                offs_ref[0, pl.ds(jj * L, L)] = o
            cp = pltpu.async_copy(vbuf.at[pl.ds(ci * 128, 128)],
                                  occ_out.at[offs_ref.at[0]], sem)
            cp.wait()
            return 0

        lax.fori_loop(0, seg // 128, chunk, 0)

    mesh = plsc.VectorSubcoreMesh(core_axis_name="c", subcore_axis_name="s")

    def call(occ_in_flat, idx_flat, vals_flat):
        return pl.kernel(
            body,
            out_type=jax.ShapeDtypeStruct((BZ * tot,), jnp.float32),
            mesh=mesh,
            scratch_types=[
                pltpu.VMEM((ch2,), jnp.float32),
                pltpu.VMEM((seg,), jnp.int32),
                pltpu.VMEM((seg,), jnp.float32),
                pltpu.VMEM((1, 128), jnp.int32),
                pltpu.SemaphoreType.DMA,
            ],
            compiler_params=pltpu.CompilerParams(needs_layout_passes=False),
        )(occ_in_flat, idx_flat, vals_flat)

    return call


def kernel(feat, W1, Wf, b1, W2, b2, W3, b3):
    b1r = b1.reshape(1, HID)
    b2r = b2.reshape(1, HID)
    w3r = W3.reshape(1, HID)
    b3r = b3.reshape(1, 1)

    r0, wp0 = LEVELS[0]
    idx65 = jnp.broadcast_to(jnp.arange(K, dtype=jnp.int32)[None], (BZ, K))
    vals = _mk_mlp_kernel(r0, 512.0 / (r0 - 1))(
        idx65.reshape(BZ, K, 1), feat, W1, Wf, b1r, W2, b2r, w3r, b3r)
    occ_flat = _mk_scatter_kernel(r0, wp0, True)(
        jnp.zeros((BZ * r0 * wp0,), jnp.float32),
        idx65.reshape(BZ * K),
        lax.optimization_barrier(vals.reshape(BZ * K)))
    occ = occ_flat.reshape(BZ, r0, wp0)

    rp, wpp = r0, wp0
    for r, wp in LEVELS[1:]:
        m, mt = _up_mats(rp, wpp, r, wp)
        occ_up, keys, stats = _mk_up_kernel(rp, wpp, r, wp)(occ, m, mt)
        idx_pad = _mk_compact_kernel(r, wp)(
            lax.optimization_barrier(keys.reshape(BZ * r * wp)),
            lax.optimization_barrier(stats.reshape(BZ, 16)))
        idx = lax.optimization_barrier(idx_pad[:BZ * K])
        vals = _mk_mlp_kernel(r, 512.0 / (r - 1))(
            idx.reshape(BZ, K, 1), feat, W1, Wf, b1r, W2, b2r, w3r, b3r)
        occ_flat = _mk_scatter_kernel(r, wp, False)(
            lax.optimization_barrier(occ_up.reshape(BZ * r * wp)), idx,
            lax.optimization_barrier(vals.reshape(BZ * K)))
        occ = occ_flat.reshape(BZ, r, wp)
        rp, wpp = r, wp

    return occ[:, :rp, :rp].reshape(BZ, 1, rp, rp)
